# baseline (speedup 1.0000x reference)
import functools

import jax
import jax.numpy as jnp
from jax import lax
from jax.experimental import pallas as pl
from jax.experimental.pallas import tpu as pltpu
from jax.experimental.pallas import tpu_sc as plsc

N = 10000
E = 160000
T = N - 1
D = 300
DP = 304
DH = 256
C = 128
N2 = 10240
NS = 16
CH = E // NS
NV = CH // 16
RED = N2 // NS
CAPF = 128
CAPH = 2048
HF = 2080
HFP = HF + 16
SELF0 = 128
XB = 6
NSEG = N * D // 16
GSZ = C * DP


def _rsqrt(x):
    i = plsc.bitcast(x, jnp.int32)
    y = plsc.bitcast(jnp.int32(0x5F3759DF) - (i >> 1), jnp.float32)
    for _ in range(3):
        y = y * (1.5 - 0.5 * x * y * y)
    return y


def _sc_body(src_l, dst_l, src_r, dst_r, x_l, x_r,
             g_out, wv_out,
             chunk, hist, mark, dinv, red, dpart, g2, xseg, xidx,
             fl_src, fl_dst, fl_slot, fl_eidx, fl_msk, fl_coef,
             feidx, fsrcv, wv, st_a, st_b, st_c, bk_e, bk_s, bk_m,
             cnt_smem,
             hist_all, dinv_sh, mark_sh, frow_e, frow_m,
             hrow_e, hrow_s, hrow_d):
    c = lax.axis_index("c")
    s = lax.axis_index("s")
    iota = lax.iota(jnp.int32, 16)
    zero16i = jnp.zeros((16,), jnp.int32)
    zero16f = jnp.zeros((16,), jnp.float32)
    ones16 = jnp.ones((16,), jnp.int32)
    is0 = s == 0

    @pl.when(is0)
    def _():
        cnt_smem[0] = 0
        cnt_smem[1] = 0

    plsc.subcore_barrier()

    @pl.when(c == 0)
    def _():
        pltpu.sync_copy(dst_l.at[pl.ds(s * CH, CH)], chunk)

    @pl.when(c == 1)
    def _():
        pltpu.sync_copy(dst_r.at[pl.ds(s * CH, CH)], chunk)

    def _zh(i, _):
        hist[pl.ds(i * 16, 16)] = zero16i
        return 0
    lax.fori_loop(0, N2 // 16, _zh, 0)

    ebase = s * CH

    def _scan1(i, _):
        d = chunk[pl.ds(i * 16, 16)]
        cnts, lastm = plsc.scan_count(d)
        plsc.addupdate_scatter(hist, [d], cnts, mask=lastm)
        tm = d == T
        npop = plsc.all_reduce_population_count(tm)[0]

        @pl.when(npop > 0)
        def _():
            row = plsc.fetch_and_add(cnt_smem.at[0], 1, subcore_id=0)

            @pl.when(row < CAPF)
            def _():
                st_a[...] = ebase + i * 16 + iota
                st_b[...] = jnp.where(tm, 1, 0)
                pltpu.sync_copy(st_a, frow_e.at[c, row])
                pltpu.sync_copy(st_b, frow_m.at[c, row])
        return 0
    lax.fori_loop(0, NV, _scan1, 0)

    pltpu.sync_copy(hist, hist_all.at[c, s])
    plsc.subcore_barrier()

    nbase = s * RED
    for r in range(NS):
        pltpu.sync_copy(hist_all.at[c, r, pl.ds(nbase, RED)], red.at[r])

    def _dred(k, _):
        acc = zero16i
        for r in range(NS):
            acc = acc + red[r, pl.ds(k * 16, 16)]
        dpart[pl.ds(k * 16, 16)] = _rsqrt((acc + 1).astype(jnp.float32))
        return 0
    lax.fori_loop(0, RED // 16, _dred, 0)
    pltpu.sync_copy(dpart, dinv_sh.at[c, pl.ds(nbase, RED)])
    plsc.subcore_barrier()

    @pl.when(is0)
    def _frontier():
        pltpu.sync_copy(dinv_sh.at[c], dinv)
        for j in range(C // 16):
            feidx[pl.ds(j * 16, 16)] = zero16i

        nfr = jnp.minimum(cnt_smem[0], CAPF)

        def _fbb(b, off):
            pltpu.sync_copy(frow_e.at[c, pl.ds(b * 32, 32)], bk_e)
            pltpu.sync_copy(frow_m.at[c, pl.ds(b * 32, 32)], bk_m)

            def _fb(j, off2):
                jv = iota * 0 + j
                ev = plsc.load_gather(bk_e, [jv, iota])
                mraw = plsc.load_gather(bk_m, [jv, iota])
                mv = (mraw > 0) & (b * 32 + j < nfr)
                ok = off2 <= C - 32

                @pl.when(ok)
                def _():
                    plsc.store_compressed(feidx.at[pl.ds(off2, 16)], ev, mask=mv)
                npop = plsc.all_reduce_population_count(mv)[0]
                return jnp.where(ok, off2 + npop, off2)
            return lax.fori_loop(0, 32, _fb, off)
        kf = lax.fori_loop(0, (nfr + 31) // 32, _fbb, jnp.int32(0))

        @pl.when(c == 0)
        def _():
            pltpu.sync_copy(src_l.at[feidx], fsrcv)

        @pl.when(c == 1)
        def _():
            pltpu.sync_copy(src_r.at[feidx], fsrcv)

        neg16 = jnp.full((16,), -1, jnp.int32)

        def _mi(i, _):
            mark[pl.ds(i * 16, 16)] = neg16
            return 0
        lax.fori_loop(0, N2 // 16, _mi, 0)

        for j in range(C // 16):
            gl = j * 16 + iota
            sv = jnp.where(gl < kf, fsrcv[pl.ds(j * 16, 16)], 0)
            plsc.store_scatter(mark, [sv], gl, mask=gl < kf)
        tvec = jnp.full((16,), T, jnp.int32)
        cvec = jnp.full((16,), C - 1, jnp.int32)
        plsc.store_scatter(mark, [tvec], cvec, mask=iota == 0)

        dinv_t = dinv[pl.ds(T - 15, 16)][15]
        for j in range(C // 16):
            wv[pl.ds(j * 16, 16)] = zero16f
        for j in range(C // 16):
            gl = j * 16 + iota
            valid = gl < kf
            sv = jnp.where(valid, fsrcv[pl.ds(j * 16, 16)], 0)
            slots = jnp.where(valid, plsc.load_gather(mark, [sv], mask=valid), 0)
            dsv = jnp.where(valid, plsc.load_gather(dinv, [sv], mask=valid), 0.0)
            cnts, lastm = plsc.scan_count(slots, mask=valid)
            plsc.addupdate_scatter(
                wv, [slots], dsv * dinv_t * cnts.astype(jnp.float32), mask=lastm)
        lastw = wv[pl.ds(C - 16, 16)]
        wv[pl.ds(C - 16, 16)] = lastw + jnp.where(
            iota == 15, dinv_t * dinv_t, 0.0)

        def _zf(i, _):
            z = zero16i
            sl = pl.ds(i * 16, 16)
            fl_src[sl] = z
            fl_dst[sl] = z
            fl_slot[sl] = z
            fl_eidx[sl] = z
            fl_msk[sl] = z
            return 0
        lax.fori_loop(0, HF // 16, _zf, 0)

        hoff = jnp.int32(0)
        for j in range(C // 16):
            gl = j * 16 + iota
            valid = gl < kf
            sv = jnp.where(valid, fsrcv[pl.ds(j * 16, 16)], 0)
            slots = jnp.where(valid, plsc.load_gather(mark, [sv], mask=valid), -1)
            canon = valid & (slots == gl)
            plsc.store_compressed(fl_src.at[pl.ds(hoff, 16)], sv, mask=canon)
            plsc.store_compressed(fl_dst.at[pl.ds(hoff, 16)], sv, mask=canon)
            plsc.store_compressed(fl_slot.at[pl.ds(hoff, 16)], slots, mask=canon)
            plsc.store_compressed(fl_msk.at[pl.ds(hoff, 16)], ones16, mask=canon)
            hoff = hoff + plsc.all_reduce_population_count(canon)[0]
        m0 = iota == 0
        plsc.store_compressed(fl_src.at[pl.ds(hoff, 16)], tvec, mask=m0)
        plsc.store_compressed(fl_dst.at[pl.ds(hoff, 16)], tvec, mask=m0)
        plsc.store_compressed(fl_slot.at[pl.ds(hoff, 16)], cvec, mask=m0)
        plsc.store_compressed(fl_msk.at[pl.ds(hoff, 16)], ones16, mask=m0)

        pltpu.sync_copy(mark, mark_sh.at[c])

    plsc.subcore_barrier()

    pltpu.sync_copy(mark_sh.at[c], mark)

    def _scan2(i, _):
        d = chunk[pl.ds(i * 16, 16)]
        slots = plsc.load_gather(mark, [d])
        hm = slots >= 0
        npop = plsc.all_reduce_population_count(hm)[0]

        @pl.when(npop > 0)
        def _():
            row = plsc.fetch_and_add(cnt_smem.at[1], 1, subcore_id=0)

            @pl.when(row < CAPH)
            def _():
                st_a[...] = ebase + i * 16 + iota
                st_b[...] = slots
                st_c[...] = d
                pltpu.sync_copy(st_a, hrow_e.at[c, row])
                pltpu.sync_copy(st_b, hrow_s.at[c, row])
                pltpu.sync_copy(st_c, hrow_d.at[c, row])
        return 0
    lax.fori_loop(0, NV, _scan2, 0)
    plsc.subcore_barrier()

    @pl.when(is0)
    def _hits():
        nhr = jnp.minimum(cnt_smem[1], CAPH)

        def _hbb(b, off):
            pltpu.sync_copy(hrow_e.at[c, pl.ds(b * 32, 32)], bk_e)
            pltpu.sync_copy(hrow_s.at[c, pl.ds(b * 32, 32)], bk_s)
            pltpu.sync_copy(hrow_d.at[c, pl.ds(b * 32, 32)], bk_m)

            def _hb(j, off2):
                jv = iota * 0 + j
                ev = plsc.load_gather(bk_e, [jv, iota])
                sl = plsc.load_gather(bk_s, [jv, iota])
                dv = plsc.load_gather(bk_m, [jv, iota])
                mv = (sl >= 0) & (b * 32 + j < nhr)
                ok = off2 <= HF - 32

                @pl.when(ok)
                def _():
                    osl = pl.ds(off2, 16)
                    plsc.store_compressed(fl_eidx.at[osl], ev, mask=mv)
                    plsc.store_compressed(fl_slot.at[osl], jnp.where(mv, sl, 0), mask=mv)
                    plsc.store_compressed(fl_dst.at[osl], dv, mask=mv)
                    plsc.store_compressed(fl_msk.at[osl], ones16, mask=mv)
                npop = plsc.all_reduce_population_count(mv)[0]
                return jnp.where(ok, off2 + npop, off2)
            return lax.fori_loop(0, 32, _hb, off)
        hend = lax.fori_loop(0, (nhr + 31) // 32, _hbb, jnp.int32(SELF0))
        hoff2 = jnp.bitwise_and(hend + 31, jnp.int32(-32))

        def _sg(b, _):
            o = SELF0 + b * 32

            @pl.when(c == 0)
            def _():
                pltpu.sync_copy(src_l.at[fl_eidx.at[pl.ds(o, 32)]],
                                fl_src.at[pl.ds(o, 32)])

            @pl.when(c == 1)
            def _():
                pltpu.sync_copy(src_r.at[fl_eidx.at[pl.ds(o, 32)]],
                                fl_src.at[pl.ds(o, 32)])
            return 0
        lax.fori_loop(0, (hoff2 - SELF0) // 32, _sg, 0)

        def _cf(j, _):
            sl = pl.ds(j * 16, 16)
            sv = fl_src[sl]
            dv = fl_dst[sl]
            mv = fl_msk[sl]
            c1 = plsc.load_gather(dinv, [sv])
            c2 = plsc.load_gather(dinv, [dv])
            fl_coef[sl] = c1 * c2 * mv.astype(jnp.float32)
            return 0
        lax.fori_loop(0, HF // 16, _cf, 0)

        def _zg(i, _):
            g2[pl.ds(i * 16, 16)] = zero16f
            return 0
        lax.fori_loop(0, GSZ // 16, _zg, 0)

        def _xb(b, _):
            o = b * XB
            for j in range(XB):
                srcj = fl_src[pl.ds(o + j, 16)][0]
                seg0 = (srcj * D) >> 4
                segs_a = jnp.minimum(seg0 + iota, NSEG - 1)
                segs_b = jnp.minimum(seg0 + 16 + iota, NSEG - 1)
                plsc.store_scatter(xidx, [j * 20 + iota], segs_a)
                plsc.store_scatter(xidx, [j * 20 + 16 + iota], segs_b,
                                   mask=iota < 4)

            @pl.when(c == 0)
            def _():
                pltpu.sync_copy(x_l.at[xidx], xseg)

            @pl.when(c == 1)
            def _():
                pltpu.sync_copy(x_r.at[xidx], xseg)

            def _row(j, _2):
                base = o + j
                coef = fl_coef[pl.ds(base, 16)][0]
                slot = fl_slot[pl.ds(base, 16)][0]
                srcj = fl_src[pl.ds(base, 16)][0]

                @pl.when(coef != 0.0)
                def _():
                    rb = slot * DP
                    shift = (srcj * D) & 15
                    fb = j * 320 + shift
                    for k in range(DP // 16):
                        col = k * 16 + iota
                        lm = col < D
                        f = fb + col
                        xv = plsc.load_gather(xseg, [f >> 4, f & 15], mask=lm)
                        plsc.addupdate_scatter(
                            g2, [rb + col], jnp.where(lm, xv * coef, 0.0),
                            mask=lm)
                return 0
            lax.fori_loop(0, XB, _row, 0)
            return 0
        lax.fori_loop(0, (hend + XB - 1) // XB, _xb, 0)

        pltpu.sync_copy(g2, g_out.at[c])
        pltpu.sync_copy(wv, wv_out.at[c])


def _tc_body(g_ref, wv_ref, w1_ref, b1_ref, w2_ref, b2_ref, wm_ref, bm_ref,
             o_ref):
    w1 = w1_ref[...]
    b1 = b1_ref[...]
    outs = []
    for cc in range(2):
        h = jnp.maximum(
            jnp.dot(g_ref[cc], w1, preferred_element_type=jnp.float32, precision=lax.Precision.HIGHEST) + b1,
            0.0)
        u = jnp.dot(wv_ref[cc:cc + 1, :], h,
                    preferred_element_type=jnp.float32,
                    precision=lax.Precision.HIGHEST)
        v = jnp.dot(u, w2_ref[...], preferred_element_type=jnp.float32,
                    precision=lax.Precision.HIGHEST) \
            + b2_ref[...]
        outs.append(v)
    vcat = jnp.concatenate(outs, axis=1)
    o_ref[...] = jnp.dot(vcat, wm_ref[...],
                         preferred_element_type=jnp.float32,
                         precision=lax.Precision.HIGHEST) + bm_ref[...]


def _sc_stage(src_l, dst_l, src_r, dst_r, x_l, x_r):
    mesh = plsc.VectorSubcoreMesh(core_axis_name="c", subcore_axis_name="s")
    sc_call = pl.kernel(
        _sc_body,
        out_type=(jax.ShapeDtypeStruct((2, GSZ), jnp.float32),
                  jax.ShapeDtypeStruct((2, C), jnp.float32)),
        mesh=mesh,
        compiler_params=pltpu.CompilerParams(needs_layout_passes=False, use_tc_tiling_on_sc=False),
        scratch_types=[
            pltpu.VMEM((CH,), jnp.int32),
            pltpu.VMEM((N2,), jnp.int32),
            pltpu.VMEM((N2,), jnp.int32),
            pltpu.VMEM((N2,), jnp.float32),
            pltpu.VMEM((NS, RED), jnp.int32),
            pltpu.VMEM((RED,), jnp.float32),
            pltpu.VMEM((GSZ,), jnp.float32),
            pltpu.VMEM((XB * 20, 16), jnp.float32),
            pltpu.VMEM((XB * 20,), jnp.int32),
            pltpu.VMEM((HF,), jnp.int32),
            pltpu.VMEM((HF,), jnp.int32),
            pltpu.VMEM((HFP,), jnp.int32),
            pltpu.VMEM((HF,), jnp.int32),
            pltpu.VMEM((HF,), jnp.int32),
            pltpu.VMEM((HFP,), jnp.float32),
            pltpu.VMEM((C,), jnp.int32),
            pltpu.VMEM((C,), jnp.int32),
            pltpu.VMEM((C,), jnp.float32),
            pltpu.VMEM((16,), jnp.int32),
            pltpu.VMEM((16,), jnp.int32),
            pltpu.VMEM((16,), jnp.int32),
            pltpu.VMEM((32, 16), jnp.int32),
            pltpu.VMEM((32, 16), jnp.int32),
            pltpu.VMEM((32, 16), jnp.int32),
            pltpu.SMEM((2,), jnp.int32),
            pltpu.HBM((2, NS, N2), jnp.int32),
            pltpu.HBM((2, N2), jnp.float32),
            pltpu.HBM((2, N2), jnp.int32),
            pltpu.HBM((2, CAPF, 16), jnp.int32),
            pltpu.HBM((2, CAPF, 16), jnp.int32),
            pltpu.HBM((2, CAPH, 16), jnp.int32),
            pltpu.HBM((2, CAPH, 16), jnp.int32),
            pltpu.HBM((2, CAPH, 16), jnp.int32),
        ],
    )
    return sc_call(src_l, dst_l, src_r, dst_r, x_l, x_r)


def kernel(x_left, edge_index_left, x_right, edge_index_right,
           W1, b1, W2, b2, Wm, bm):
    g_flat, wvec = _sc_stage(edge_index_left[0], edge_index_left[1],
                             edge_index_right[0], edge_index_right[1],
                             x_left.reshape(NSEG, 16),
                             x_right.reshape(NSEG, 16))
    g3 = g_flat.reshape(2, C, DP)
    w1p = jnp.concatenate([W1, jnp.zeros((DP - D, DH), W1.dtype)], axis=0)
    out = pl.pallas_call(
        _tc_body,
        out_shape=jax.ShapeDtypeStruct((1, 2), jnp.float32),
    )(g3, wvec, w1p, b1.reshape(1, DH), W2, b2.reshape(1, DH),
      Wm, bm.reshape(1, 2))
    return out


# speedup vs baseline: 1.1949x; 1.1949x over previous
import functools

import jax
import jax.numpy as jnp
from jax import lax
from jax.experimental import pallas as pl
from jax.experimental.pallas import tpu as pltpu
from jax.experimental.pallas import tpu_sc as plsc

N = 10000
E = 160000
T = N - 1
D = 300
DP = 304
DH = 256
C = 128
N2 = 10240
NS = 16
CH = E // NS
NV = CH // 16
RED = N2 // NS
CAPF = 128
CAPH = 2048
HF = 2080
HFP = HF + 16
SELF0 = 128
XB = 6
NSEG = N * D // 16
GSZ = C * DP


def _rsqrt(x):
    i = plsc.bitcast(x, jnp.int32)
    y = plsc.bitcast(jnp.int32(0x5F3759DF) - (i >> 1), jnp.float32)
    for _ in range(3):
        y = y * (1.5 - 0.5 * x * y * y)
    return y


def _sc_body(src_l, dst_l, src_r, dst_r, x_l, x_r,
             g_out, wv_out,
             chunk, hist, mark, dinv, red, dpart, g2, xseg, xidx,
             fl_src, fl_dst, fl_slot, fl_eidx, fl_msk, fl_coef,
             feidx, fsrcv, wv, st_a, st_b, st_c, bk_e, bk_s, bk_m,
             gred, gacc,
             cnt_smem,
             hist_all, dinv_sh, mark_sh, frow_e, frow_m,
             hrow_e, hrow_s, hrow_d,
             flsrc_sh, flslot_sh, flcoef_sh, hend_sh, gpart):
    c = lax.axis_index("c")
    s = lax.axis_index("s")
    iota = lax.iota(jnp.int32, 16)
    zero16i = jnp.zeros((16,), jnp.int32)
    zero16f = jnp.zeros((16,), jnp.float32)
    ones16 = jnp.ones((16,), jnp.int32)
    is0 = s == 0

    @pl.when(is0)
    def _():
        cnt_smem[0] = 0
        cnt_smem[1] = 0

    plsc.subcore_barrier()

    @pl.when(c == 0)
    def _():
        pltpu.sync_copy(dst_l.at[pl.ds(s * CH, CH)], chunk)

    @pl.when(c == 1)
    def _():
        pltpu.sync_copy(dst_r.at[pl.ds(s * CH, CH)], chunk)

    def _zh(i, _):
        hist[pl.ds(i * 16, 16)] = zero16i
        return 0
    lax.fori_loop(0, N2 // 16, _zh, 0)

    ebase = s * CH

    def _scan1(i, _):
        d = chunk[pl.ds(i * 16, 16)]
        cnts, lastm = plsc.scan_count(d)
        plsc.addupdate_scatter(hist, [d], cnts, mask=lastm)
        tm = d == T
        npop = plsc.all_reduce_population_count(tm)[0]

        @pl.when(npop > 0)
        def _():
            row = plsc.fetch_and_add(cnt_smem.at[0], 1, subcore_id=0)

            @pl.when(row < CAPF)
            def _():
                st_a[...] = ebase + i * 16 + iota
                st_b[...] = jnp.where(tm, 1, 0)
                pltpu.sync_copy(st_a, frow_e.at[c, row])
                pltpu.sync_copy(st_b, frow_m.at[c, row])
        return 0
    lax.fori_loop(0, NV, _scan1, 0)

    pltpu.sync_copy(hist, hist_all.at[c, s])
    plsc.subcore_barrier()

    nbase = s * RED
    for r in range(NS):
        pltpu.sync_copy(hist_all.at[c, r, pl.ds(nbase, RED)], red.at[r])

    def _dred(k, _):
        acc = zero16i
        for r in range(NS):
            acc = acc + red[r, pl.ds(k * 16, 16)]
        dpart[pl.ds(k * 16, 16)] = _rsqrt((acc + 1).astype(jnp.float32))
        return 0
    lax.fori_loop(0, RED // 16, _dred, 0)
    pltpu.sync_copy(dpart, dinv_sh.at[c, pl.ds(nbase, RED)])
    plsc.subcore_barrier()

    @pl.when(is0)
    def _frontier():
        pltpu.sync_copy(dinv_sh.at[c], dinv)
        for j in range(C // 16):
            feidx[pl.ds(j * 16, 16)] = zero16i

        nfr = jnp.minimum(cnt_smem[0], CAPF)

        def _fbb(b, off):
            pltpu.sync_copy(frow_e.at[c, pl.ds(b * 32, 32)], bk_e)
            pltpu.sync_copy(frow_m.at[c, pl.ds(b * 32, 32)], bk_m)

            def _fb(j, off2):
                jv = iota * 0 + j
                ev = plsc.load_gather(bk_e, [jv, iota])
                mraw = plsc.load_gather(bk_m, [jv, iota])
                mv = (mraw > 0) & (b * 32 + j < nfr)
                ok = off2 <= C - 32

                @pl.when(ok)
                def _():
                    plsc.store_compressed(feidx.at[pl.ds(off2, 16)], ev, mask=mv)
                npop = plsc.all_reduce_population_count(mv)[0]
                return jnp.where(ok, off2 + npop, off2)
            return lax.fori_loop(0, 32, _fb, off)
        kf = lax.fori_loop(0, (nfr + 31) // 32, _fbb, jnp.int32(0))

        @pl.when(c == 0)
        def _():
            pltpu.sync_copy(src_l.at[feidx], fsrcv)

        @pl.when(c == 1)
        def _():
            pltpu.sync_copy(src_r.at[feidx], fsrcv)

        neg16 = jnp.full((16,), -1, jnp.int32)

        def _mi(i, _):
            mark[pl.ds(i * 16, 16)] = neg16
            return 0
        lax.fori_loop(0, N2 // 16, _mi, 0)

        for j in range(C // 16):
            gl = j * 16 + iota
            sv = jnp.where(gl < kf, fsrcv[pl.ds(j * 16, 16)], 0)
            plsc.store_scatter(mark, [sv], gl, mask=gl < kf)
        tvec = jnp.full((16,), T, jnp.int32)
        cvec = jnp.full((16,), C - 1, jnp.int32)
        plsc.store_scatter(mark, [tvec], cvec, mask=iota == 0)

        dinv_t = dinv[pl.ds(T - 15, 16)][15]
        for j in range(C // 16):
            wv[pl.ds(j * 16, 16)] = zero16f
        for j in range(C // 16):
            gl = j * 16 + iota
            valid = gl < kf
            sv = jnp.where(valid, fsrcv[pl.ds(j * 16, 16)], 0)
            slots = jnp.where(valid, plsc.load_gather(mark, [sv], mask=valid), 0)
            dsv = jnp.where(valid, plsc.load_gather(dinv, [sv], mask=valid), 0.0)
            cnts, lastm = plsc.scan_count(slots, mask=valid)
            plsc.addupdate_scatter(
                wv, [slots], dsv * dinv_t * cnts.astype(jnp.float32), mask=lastm)
        lastw = wv[pl.ds(C - 16, 16)]
        wv[pl.ds(C - 16, 16)] = lastw + jnp.where(
            iota == 15, dinv_t * dinv_t, 0.0)

        def _zf(i, _):
            z = zero16i
            sl = pl.ds(i * 16, 16)
            fl_src[sl] = z
            fl_dst[sl] = z
            fl_slot[sl] = z
            fl_eidx[sl] = z
            fl_msk[sl] = z
            return 0
        lax.fori_loop(0, HF // 16, _zf, 0)

        hoff = jnp.int32(0)
        for j in range(C // 16):
            gl = j * 16 + iota
            valid = gl < kf
            sv = jnp.where(valid, fsrcv[pl.ds(j * 16, 16)], 0)
            slots = jnp.where(valid, plsc.load_gather(mark, [sv], mask=valid), -1)
            canon = valid & (slots == gl)
            plsc.store_compressed(fl_src.at[pl.ds(hoff, 16)], sv, mask=canon)
            plsc.store_compressed(fl_dst.at[pl.ds(hoff, 16)], sv, mask=canon)
            plsc.store_compressed(fl_slot.at[pl.ds(hoff, 16)], slots, mask=canon)
            plsc.store_compressed(fl_msk.at[pl.ds(hoff, 16)], ones16, mask=canon)
            hoff = hoff + plsc.all_reduce_population_count(canon)[0]
        m0 = iota == 0
        plsc.store_compressed(fl_src.at[pl.ds(hoff, 16)], tvec, mask=m0)
        plsc.store_compressed(fl_dst.at[pl.ds(hoff, 16)], tvec, mask=m0)
        plsc.store_compressed(fl_slot.at[pl.ds(hoff, 16)], cvec, mask=m0)
        plsc.store_compressed(fl_msk.at[pl.ds(hoff, 16)], ones16, mask=m0)

        pltpu.sync_copy(mark, mark_sh.at[c])

    plsc.subcore_barrier()

    pltpu.sync_copy(mark_sh.at[c], mark)

    def _scan2(i, _):
        d = chunk[pl.ds(i * 16, 16)]
        slots = plsc.load_gather(mark, [d])
        hm = slots >= 0
        npop = plsc.all_reduce_population_count(hm)[0]

        @pl.when(npop > 0)
        def _():
            row = plsc.fetch_and_add(cnt_smem.at[1], 1, subcore_id=0)

            @pl.when(row < CAPH)
            def _():
                st_a[...] = ebase + i * 16 + iota
                st_b[...] = slots
                st_c[...] = d
                pltpu.sync_copy(st_a, hrow_e.at[c, row])
                pltpu.sync_copy(st_b, hrow_s.at[c, row])
                pltpu.sync_copy(st_c, hrow_d.at[c, row])
        return 0
    lax.fori_loop(0, NV, _scan2, 0)
    plsc.subcore_barrier()

    @pl.when(is0)
    def _hits():
        nhr = jnp.minimum(cnt_smem[1], CAPH)

        def _hbb(b, off):
            pltpu.sync_copy(hrow_e.at[c, pl.ds(b * 32, 32)], bk_e)
            pltpu.sync_copy(hrow_s.at[c, pl.ds(b * 32, 32)], bk_s)
            pltpu.sync_copy(hrow_d.at[c, pl.ds(b * 32, 32)], bk_m)

            def _hb(j, off2):
                jv = iota * 0 + j
                ev = plsc.load_gather(bk_e, [jv, iota])
                sl = plsc.load_gather(bk_s, [jv, iota])
                dv = plsc.load_gather(bk_m, [jv, iota])
                mv = (sl >= 0) & (b * 32 + j < nhr)
                ok = off2 <= HF - 32

                @pl.when(ok)
                def _():
                    osl = pl.ds(off2, 16)
                    plsc.store_compressed(fl_eidx.at[osl], ev, mask=mv)
                    plsc.store_compressed(fl_slot.at[osl], jnp.where(mv, sl, 0), mask=mv)
                    plsc.store_compressed(fl_dst.at[osl], dv, mask=mv)
                    plsc.store_compressed(fl_msk.at[osl], ones16, mask=mv)
                npop = plsc.all_reduce_population_count(mv)[0]
                return jnp.where(ok, off2 + npop, off2)
            return lax.fori_loop(0, 32, _hb, off)
        hend = lax.fori_loop(0, (nhr + 31) // 32, _hbb, jnp.int32(SELF0))
        hoff2 = jnp.bitwise_and(hend + 31, jnp.int32(-32))

        def _sg(b, _):
            o = SELF0 + b * 32

            @pl.when(c == 0)
            def _():
                pltpu.sync_copy(src_l.at[fl_eidx.at[pl.ds(o, 32)]],
                                fl_src.at[pl.ds(o, 32)])

            @pl.when(c == 1)
            def _():
                pltpu.sync_copy(src_r.at[fl_eidx.at[pl.ds(o, 32)]],
                                fl_src.at[pl.ds(o, 32)])
            return 0
        lax.fori_loop(0, (hoff2 - SELF0) // 32, _sg, 0)

        def _cf(j, _):
            sl = pl.ds(j * 16, 16)
            sv = fl_src[sl]
            dv = fl_dst[sl]
            mv = fl_msk[sl]
            c1 = plsc.load_gather(dinv, [sv])
            c2 = plsc.load_gather(dinv, [dv])
            fl_coef[sl] = c1 * c2 * mv.astype(jnp.float32)
            return 0
        lax.fori_loop(0, HF // 16, _cf, 0)

        pltpu.sync_copy(fl_src, flsrc_sh.at[c])
        pltpu.sync_copy(fl_slot, flslot_sh.at[c])
        pltpu.sync_copy(fl_coef, flcoef_sh.at[c])
        st_a[...] = zero16i + hend
        pltpu.sync_copy(st_a, hend_sh.at[c])
        pltpu.sync_copy(wv, wv_out.at[c])

    plsc.subcore_barrier()

    pltpu.sync_copy(flsrc_sh.at[c], fl_src)
    pltpu.sync_copy(flslot_sh.at[c], fl_slot)
    pltpu.sync_copy(flcoef_sh.at[c], fl_coef)
    pltpu.sync_copy(hend_sh.at[c], st_a)
    hend2 = st_a[pl.ds(0, 16)][0]

    def _zg(i, _):
        g2[pl.ds(i * 16, 16)] = zero16f
        return 0
    lax.fori_loop(0, GSZ // 16, _zg, 0)

    nbat = (hend2 + XB - 1) // XB
    mybat = jnp.maximum((nbat - s + NS - 1) // NS, 0)

    def _xb(k, _):
        o = (k * NS + s) * XB
        for j in range(XB):
            srcj = fl_src[pl.ds(o + j, 16)][0]
            seg0 = (srcj * D) >> 4
            segs_a = jnp.minimum(seg0 + iota, NSEG - 1)
            segs_b = jnp.minimum(seg0 + 16 + iota, NSEG - 1)
            plsc.store_scatter(xidx, [j * 20 + iota], segs_a)
            plsc.store_scatter(xidx, [j * 20 + 16 + iota], segs_b,
                               mask=iota < 4)

        @pl.when(c == 0)
        def _():
            pltpu.sync_copy(x_l.at[xidx], xseg)

        @pl.when(c == 1)
        def _():
            pltpu.sync_copy(x_r.at[xidx], xseg)

        def _row(j, _2):
            base = o + j
            coef = fl_coef[pl.ds(base, 16)][0]
            slot = fl_slot[pl.ds(base, 16)][0]
            srcj = fl_src[pl.ds(base, 16)][0]

            @pl.when(coef != 0.0)
            def _():
                rb = slot * DP
                shift = (srcj * D) & 15
                fb = j * 320 + shift
                for kk in range(DP // 16):
                    col = kk * 16 + iota
                    lm = col < D
                    f = fb + col
                    xv = plsc.load_gather(xseg, [f >> 4, f & 15], mask=lm)
                    plsc.addupdate_scatter(
                        g2, [rb + col], jnp.where(lm, xv * coef, 0.0),
                        mask=lm)
            return 0
        lax.fori_loop(0, XB, _row, 0)
        return 0
    lax.fori_loop(0, mybat, _xb, 0)

    pltpu.sync_copy(g2, gpart.at[c, s])
    plsc.subcore_barrier()

    gbase = s * (GSZ // NS)

    def _zr(i, _):
        gacc[pl.ds(i * 16, 16)] = zero16f
        return 0
    lax.fori_loop(0, GSZ // NS // 16, _zr, 0)
    for r in range(NS):
        pltpu.sync_copy(gpart.at[c, r, pl.ds(gbase, GSZ // NS)], gred)

        def _ar(i, _):
            sl = pl.ds(i * 16, 16)
            gacc[sl] = gacc[sl] + gred[sl]
            return 0
        lax.fori_loop(0, GSZ // NS // 16, _ar, 0)
    pltpu.sync_copy(gacc, g_out.at[c, pl.ds(gbase, GSZ // NS)])


def _tc_body(g_ref, wv_ref, w1_ref, b1_ref, w2_ref, b2_ref, wm_ref, bm_ref,
             o_ref):
    w1 = w1_ref[...]
    b1 = b1_ref[...]
    outs = []
    for cc in range(2):
        h = jnp.maximum(
            jnp.dot(g_ref[cc], w1, preferred_element_type=jnp.float32, precision=lax.Precision.HIGHEST) + b1,
            0.0)
        u = jnp.dot(wv_ref[cc:cc + 1, :], h,
                    preferred_element_type=jnp.float32,
                    precision=lax.Precision.HIGHEST)
        v = jnp.dot(u, w2_ref[...], preferred_element_type=jnp.float32,
                    precision=lax.Precision.HIGHEST) \
            + b2_ref[...]
        outs.append(v)
    vcat = jnp.concatenate(outs, axis=1)
    o_ref[...] = jnp.dot(vcat, wm_ref[...],
                         preferred_element_type=jnp.float32,
                         precision=lax.Precision.HIGHEST) + bm_ref[...]


def _sc_stage(src_l, dst_l, src_r, dst_r, x_l, x_r):
    mesh = plsc.VectorSubcoreMesh(core_axis_name="c", subcore_axis_name="s")
    sc_call = pl.kernel(
        _sc_body,
        out_type=(jax.ShapeDtypeStruct((2, GSZ), jnp.float32),
                  jax.ShapeDtypeStruct((2, C), jnp.float32)),
        mesh=mesh,
        compiler_params=pltpu.CompilerParams(needs_layout_passes=False, use_tc_tiling_on_sc=False),
        scratch_types=[
            pltpu.VMEM((CH,), jnp.int32),
            pltpu.VMEM((N2,), jnp.int32),
            pltpu.VMEM((N2,), jnp.int32),
            pltpu.VMEM((N2,), jnp.float32),
            pltpu.VMEM((NS, RED), jnp.int32),
            pltpu.VMEM((RED,), jnp.float32),
            pltpu.VMEM((GSZ,), jnp.float32),
            pltpu.VMEM((XB * 20, 16), jnp.float32),
            pltpu.VMEM((XB * 20,), jnp.int32),
            pltpu.VMEM((HF,), jnp.int32),
            pltpu.VMEM((HF,), jnp.int32),
            pltpu.VMEM((HFP,), jnp.int32),
            pltpu.VMEM((HF,), jnp.int32),
            pltpu.VMEM((HF,), jnp.int32),
            pltpu.VMEM((HFP,), jnp.float32),
            pltpu.VMEM((C,), jnp.int32),
            pltpu.VMEM((C,), jnp.int32),
            pltpu.VMEM((C,), jnp.float32),
            pltpu.VMEM((16,), jnp.int32),
            pltpu.VMEM((16,), jnp.int32),
            pltpu.VMEM((16,), jnp.int32),
            pltpu.VMEM((32, 16), jnp.int32),
            pltpu.VMEM((32, 16), jnp.int32),
            pltpu.VMEM((32, 16), jnp.int32),
            pltpu.VMEM((GSZ // NS,), jnp.float32),
            pltpu.VMEM((GSZ // NS,), jnp.float32),
            pltpu.SMEM((2,), jnp.int32),
            pltpu.HBM((2, NS, N2), jnp.int32),
            pltpu.HBM((2, N2), jnp.float32),
            pltpu.HBM((2, N2), jnp.int32),
            pltpu.HBM((2, CAPF, 16), jnp.int32),
            pltpu.HBM((2, CAPF, 16), jnp.int32),
            pltpu.HBM((2, CAPH, 16), jnp.int32),
            pltpu.HBM((2, CAPH, 16), jnp.int32),
            pltpu.HBM((2, CAPH, 16), jnp.int32),
            pltpu.HBM((2, HF), jnp.int32),
            pltpu.HBM((2, HFP), jnp.int32),
            pltpu.HBM((2, HFP), jnp.float32),
            pltpu.HBM((2, 16), jnp.int32),
            pltpu.HBM((2, NS, GSZ), jnp.float32),
        ],
    )
    return sc_call(src_l, dst_l, src_r, dst_r, x_l, x_r)


def kernel(x_left, edge_index_left, x_right, edge_index_right,
           W1, b1, W2, b2, Wm, bm):
    g_flat, wvec = _sc_stage(edge_index_left[0], edge_index_left[1],
                             edge_index_right[0], edge_index_right[1],
                             x_left.reshape(NSEG, 16),
                             x_right.reshape(NSEG, 16))
    g3 = g_flat.reshape(2, C, DP)
    w1p = jnp.concatenate([W1, jnp.zeros((DP - D, DH), W1.dtype)], axis=0)
    out = pl.pallas_call(
        _tc_body,
        out_shape=jax.ShapeDtypeStruct((1, 2), jnp.float32),
    )(g3, wvec, w1p, b1.reshape(1, DH), W2, b2.reshape(1, DH),
      Wm, bm.reshape(1, 2))
    return out
